# baseline (speedup 1.0000x reference)
import functools

import jax
import jax.numpy as jnp
from jax import lax
from jax.experimental import pallas as pl
from jax.experimental.pallas import tpu as pltpu
from jax.experimental.pallas import tpu_sc as plsc

RES_ = 128
GRID_ = RES_ * RES_ * RES_
NC_ = 2
NS_ = 16
LANES_ = 16
HALF_ = GRID_ // NC_
SUB_ = 1024
NVEC_ = SUB_ // LANES_


def _voxelize(p):
    u = (p + 1.0) * jnp.float32(0.5) * jnp.float32(RES_)
    i = u.astype(jnp.int32)
    return jnp.clip(i, 0, RES_ - 1)


def _sc_body(n_total, pos_hbm, imp_hbm, gv_hbm, out_hbm,
             posbuf, impbuf, valbuf, zbuf, grid_sh, sem):
    c = lax.axis_index("c")
    s = lax.axis_index("s")
    chunk = n_total // NS_
    iters = chunk // SUB_
    half_lo = c * HALF_
    lane = jnp.arange(LANES_, dtype=jnp.int32)
    lane3 = lane * 3

    seg = HALF_ // NS_
    pltpu.sync_copy(gv_hbm.at[pl.ds(half_lo + s * seg, seg)],
                    grid_sh.at[pl.ds(s * seg, seg)])

    @pl.when(s == 0)
    def _init_dump():
        for j in range(128 // LANES_):
            zbuf[pl.ds(j * LANES_, LANES_)] = jnp.zeros((LANES_,), jnp.float32)
        pltpu.sync_copy(zbuf, grid_sh.at[pl.ds(HALF_, 128)])

    plsc.subcore_barrier()

    chunkbase = s * chunk

    def local_index(k):
        off = k * (3 * LANES_)
        x = plsc.load_gather(posbuf, [lane3 + off])
        y = plsc.load_gather(posbuf, [lane3 + (off + 1)])
        z = plsc.load_gather(posbuf, [lane3 + (off + 2)])
        lin = (_voxelize(x) * RES_ + _voxelize(y)) * RES_ + _voxelize(z)
        keep = (lin >= half_lo) & (lin < half_lo + HALF_)
        return jnp.where(keep, lin - half_lo, HALF_), keep

    @pl.loop(0, iters)
    def _p1(t):
        base = chunkbase + t * SUB_
        pltpu.sync_copy(pos_hbm.at[pl.ds(base * 3, SUB_ * 3)], posbuf)
        pltpu.sync_copy(imp_hbm.at[pl.ds(base, SUB_)], impbuf)
        copies = []
        for k in range(NVEC_):
            loc, _ = local_index(k)
            copies.append(pltpu.async_copy(
                impbuf.at[pl.ds(k * LANES_, LANES_)],
                grid_sh.at[loc], sem, add=True))
        for cp in copies:
            cp.wait()

    plsc.subcore_barrier()

    @pl.loop(0, iters)
    def _p2(t):
        base = chunkbase + t * SUB_
        pltpu.sync_copy(pos_hbm.at[pl.ds(base * 3, SUB_ * 3)], posbuf)
        keeps = []
        copies = []
        for k in range(NVEC_):
            loc, keep = local_index(k)
            keeps.append(keep)
            copies.append(pltpu.async_copy(
                grid_sh.at[loc], valbuf.at[pl.ds(k * LANES_, LANES_)], sem))
        for cp in copies:
            cp.wait()
        copies = []
        for k in range(NVEC_):
            gidx = jnp.where(keeps[k], base + k * LANES_ + lane,
                             n_total + c * 128)
            copies.append(pltpu.async_copy(
                valbuf.at[pl.ds(k * LANES_, LANES_)],
                out_hbm.at[gidx], sem))
        for cp in copies:
            cp.wait()


def kernel(positions, importances, grid_values):
    n = positions.shape[0]
    pos_flat = positions.reshape(-1)
    imp_flat = importances.reshape(-1)
    gv = grid_values.reshape(-1)
    mesh = plsc.VectorSubcoreMesh(core_axis_name="c", subcore_axis_name="s",
                                  num_cores=NC_, num_subcores=NS_)
    out = pl.kernel(
        functools.partial(_sc_body, n),
        out_type=jax.ShapeDtypeStruct((n + 256,), jnp.float32),
        mesh=mesh,
        compiler_params=pltpu.CompilerParams(needs_layout_passes=False),
        scratch_types=[
            pltpu.VMEM((SUB_ * 3,), jnp.float32),
            pltpu.VMEM((SUB_,), jnp.float32),
            pltpu.VMEM((SUB_,), jnp.float32),
            pltpu.VMEM((128,), jnp.float32),
            pltpu.VMEM_SHARED((HALF_ + 128,), jnp.float32),
            pltpu.SemaphoreType.DMA,
        ],
    )(pos_flat, imp_flat, gv)
    return out[:n].reshape(n, 1)


# speedup vs baseline: 23.6727x; 23.6727x over previous
import functools

import jax
import jax.numpy as jnp
from jax import lax
from jax.experimental import pallas as pl
from jax.experimental.pallas import tpu as pltpu
from jax.experimental.pallas import tpu_sc as plsc

RES_ = 128
GRID_ = RES_ * RES_ * RES_
NC_ = 2
NS_ = 16
LANES_ = 16
HALF_ = GRID_ // NC_
TILEG_ = HALF_ // NS_
P_ = 512
NVEC_ = P_ // LANES_


def _voxelize(p):
    u = (p + 1.0) * jnp.float32(0.5) * jnp.float32(RES_)
    i = u.astype(jnp.int32)
    return jnp.clip(i, 0, RES_ - 1)


def _sc_body(n_total, pos_hbm, imp_hbm, gv_hbm, out_hbm,
             posbuf, prodidx, considx, consval, resp16, outbuf, subgrid,
             sp_idx, sp_val, sp_resp, sem):
    c = lax.axis_index("c")
    s = lax.axis_index("s")
    chunk = n_total // NS_
    rounds = chunk // P_
    half_lo = c * HALF_
    lane = jnp.arange(LANES_, dtype=jnp.int32)
    lane3 = lane * 3

    pltpu.sync_copy(gv_hbm.at[pl.ds(half_lo + s * TILEG_, TILEG_)],
                    subgrid.at[pl.ds(0, TILEG_)])
    subgrid[pl.ds(TILEG_, LANES_)] = jnp.zeros((LANES_,), jnp.float32)
    plsc.subcore_barrier()

    chunkbase = s * chunk

    def produce_indices(base):
        pltpu.sync_copy(pos_hbm.at[pl.ds(base * 3, P_ * 3)], posbuf)

        @pl.loop(0, NVEC_)
        def _vec(j):
            off = j * (3 * LANES_)
            x = plsc.load_gather(posbuf, [lane3 + off])
            y = plsc.load_gather(posbuf, [lane3 + (off + 1)])
            z = plsc.load_gather(posbuf, [lane3 + (off + 2)])
            lin = (_voxelize(x) * RES_ + _voxelize(y)) * RES_ + _voxelize(z)
            keep = (lin >= half_lo) & (lin < half_lo + HALF_)
            plsc.store_scatter(prodidx, [lane + j * LANES_],
                               jnp.where(keep, lin - half_lo, HALF_))
        pltpu.sync_copy(prodidx, sp_idx.at[pl.ds(s * P_, P_)])

    @pl.loop(0, rounds)
    def _p1(r):
        base = chunkbase + r * P_
        pltpu.sync_copy(imp_hbm.at[pl.ds(base, P_)],
                        sp_val.at[pl.ds(s * P_, P_)])
        produce_indices(base)
        plsc.subcore_barrier()

        pltpu.sync_copy(sp_idx, considx)
        pltpu.sync_copy(sp_val, consval)

        @pl.loop(0, NS_ * NVEC_)
        def _add(j):
            vidx = plsc.load_gather(considx, [lane + j * LANES_])
            vval = plsc.load_gather(consval, [lane + j * LANES_])
            own = (vidx >> 16) == s
            plsc.addupdate_scatter(subgrid, [vidx & (TILEG_ - 1)], vval,
                                   mask=own)
        plsc.subcore_barrier()

    @pl.loop(0, rounds)
    def _p2(r):
        base = chunkbase + r * P_
        produce_indices(base)
        plsc.subcore_barrier()

        pltpu.sync_copy(sp_idx, considx)

        @pl.loop(0, NS_ * NVEC_)
        def _ans(j):
            vidx = plsc.load_gather(considx, [lane + j * LANES_])
            own = (vidx >> 16) == s
            lidx = jnp.where(own, vidx & (TILEG_ - 1), TILEG_)
            vals = plsc.load_gather(subgrid, [lidx])
            plsc.store_scatter(consval, [lane + j * LANES_], vals)
        copies = []
        for p in range(NS_):
            copies.append(pltpu.async_copy(
                consval.at[pl.ds(p * P_, P_)],
                sp_resp.at[pl.ds((p * NS_ + s) * P_, P_)], sem))
        for cp in copies:
            cp.wait()
        plsc.subcore_barrier()

        pltpu.sync_copy(sp_resp.at[pl.ds(s * NS_ * P_, NS_ * P_)], resp16)

        @pl.loop(0, NVEC_)
        def _comb(j):
            vidx = plsc.load_gather(prodidx, [lane + j * LANES_])
            dvec = vidx >> 16
            acc = jnp.zeros((LANES_,), jnp.float32)
            for d in range(NS_):
                rv = plsc.load_gather(resp16, [lane + (d * P_ + j * LANES_)])
                acc = jnp.where(dvec == d, rv, acc)
            plsc.store_scatter(outbuf, [lane + j * LANES_], acc)
        pltpu.sync_copy(outbuf, out_hbm.at[c, pl.ds(base, P_)])
        plsc.subcore_barrier()


def kernel(positions, importances, grid_values):
    n = positions.shape[0]
    pos_flat = positions.reshape(-1)
    imp_flat = importances.reshape(-1)
    gv = grid_values.reshape(-1)
    mesh = plsc.VectorSubcoreMesh(core_axis_name="c", subcore_axis_name="s",
                                  num_cores=NC_, num_subcores=NS_)
    out = pl.kernel(
        functools.partial(_sc_body, n),
        out_type=jax.ShapeDtypeStruct((NC_, n), jnp.float32),
        mesh=mesh,
        compiler_params=pltpu.CompilerParams(needs_layout_passes=False),
        scratch_types=[
            pltpu.VMEM((P_ * 3,), jnp.float32),
            pltpu.VMEM((P_,), jnp.int32),
            pltpu.VMEM((NS_ * P_,), jnp.int32),
            pltpu.VMEM((NS_ * P_,), jnp.float32),
            pltpu.VMEM((NS_ * P_,), jnp.float32),
            pltpu.VMEM((P_,), jnp.float32),
            pltpu.VMEM((TILEG_ + LANES_,), jnp.float32),
            pltpu.VMEM_SHARED((NS_ * P_,), jnp.int32),
            pltpu.VMEM_SHARED((NS_ * P_,), jnp.float32),
            pltpu.VMEM_SHARED((NS_ * NS_ * P_,), jnp.float32),
            pltpu.SemaphoreType.DMA,
        ],
    )(pos_flat, imp_flat, gv)
    return (out[0] + out[1]).reshape(n, 1)


# speedup vs baseline: 24.3332x; 1.0279x over previous
import functools

import jax
import jax.numpy as jnp
from jax import lax
from jax.experimental import pallas as pl
from jax.experimental.pallas import tpu as pltpu
from jax.experimental.pallas import tpu_sc as plsc

RES_ = 128
GRID_ = RES_ * RES_ * RES_
NC_ = 2
NS_ = 16
LANES_ = 16
HALF_ = GRID_ // NC_
TILEG_ = HALF_ // NS_
P_ = 512
NVEC_ = P_ // LANES_
RP_ = NS_ * P_


def _voxelize(p):
    u = (p + 1.0) * jnp.float32(0.5) * jnp.float32(RES_)
    i = u.astype(jnp.int32)
    return jnp.clip(i, 0, RES_ - 1)


def _sc_body(n_total, pos_hbm, imp_hbm, gv_hbm, out_hbm, idxscr_hbm,
             posbuf, prodidx, considx, consval, resp16, outbuf, subgrid,
             sp_idx, sp_val, sp_resp, sem):
    c = lax.axis_index("c")
    s = lax.axis_index("s")
    chunk = n_total // NS_
    rounds = chunk // P_
    half_lo = c * HALF_
    lane = jnp.arange(LANES_, dtype=jnp.int32)
    lane3 = lane * 3

    pltpu.sync_copy(gv_hbm.at[pl.ds(half_lo + s * TILEG_, TILEG_)],
                    subgrid.at[pl.ds(0, TILEG_)])
    subgrid[pl.ds(TILEG_, LANES_)] = jnp.zeros((LANES_,), jnp.float32)
    plsc.subcore_barrier()

    chunkbase = s * chunk

    @pl.loop(0, rounds)
    def _p1(r):
        par = (r % 2) * RP_
        base = chunkbase + r * P_
        cp_imp = pltpu.async_copy(imp_hbm.at[pl.ds(base, P_)],
                                  sp_val.at[pl.ds(par + s * P_, P_)], sem)
        cp_pos = pltpu.async_copy(pos_hbm.at[pl.ds(base * 3, P_ * 3)],
                                  posbuf, sem)
        cp_pos.wait()

        @pl.loop(0, NVEC_)
        def _vec(j):
            off = j * (3 * LANES_)
            x = plsc.load_gather(posbuf, [lane3 + off])
            y = plsc.load_gather(posbuf, [lane3 + (off + 1)])
            z = plsc.load_gather(posbuf, [lane3 + (off + 2)])
            lin = (_voxelize(x) * RES_ + _voxelize(y)) * RES_ + _voxelize(z)
            keep = (lin >= half_lo) & (lin < half_lo + HALF_)
            plsc.store_scatter(prodidx, [lane + j * LANES_],
                               jnp.where(keep, lin - half_lo, HALF_))

        cp_scr = pltpu.async_copy(
            prodidx, idxscr_hbm.at[c, pl.ds(r * RP_ + s * P_, P_)], sem)
        pltpu.sync_copy(prodidx, sp_idx.at[pl.ds(par + s * P_, P_)])
        cp_scr.wait()
        cp_imp.wait()
        plsc.subcore_barrier()

        pltpu.sync_copy(sp_idx.at[pl.ds(par, RP_)], considx)
        pltpu.sync_copy(sp_val.at[pl.ds(par, RP_)], consval)

        @pl.loop(0, NS_ * NVEC_)
        def _add(j):
            vidx = plsc.load_gather(considx, [lane + j * LANES_])
            vval = plsc.load_gather(consval, [lane + j * LANES_])
            own = (vidx >> 16) == s
            plsc.addupdate_scatter(subgrid, [vidx & (TILEG_ - 1)], vval,
                                   mask=own)

    plsc.subcore_barrier()

    @pl.loop(0, rounds)
    def _p2(r):
        par = (r % 2) * (NS_ * RP_)
        base = chunkbase + r * P_
        pltpu.sync_copy(idxscr_hbm.at[c, pl.ds(r * RP_, RP_)], considx)

        @pl.loop(0, NS_ * NVEC_)
        def _ans(j):
            vidx = plsc.load_gather(considx, [lane + j * LANES_])
            own = (vidx >> 16) == s
            lidx = jnp.where(own, vidx & (TILEG_ - 1), TILEG_)
            vals = plsc.load_gather(subgrid, [lidx])
            plsc.store_scatter(consval, [lane + j * LANES_], vals)

        copies = []
        for p in range(NS_):
            copies.append(pltpu.async_copy(
                consval.at[pl.ds(p * P_, P_)],
                sp_resp.at[pl.ds(par + (p * NS_ + s) * P_, P_)], sem))
        for cp in copies:
            cp.wait()
        plsc.subcore_barrier()

        pltpu.sync_copy(sp_resp.at[pl.ds(par + s * NS_ * P_, NS_ * P_)],
                        resp16.at[pl.ds(0, NS_ * P_)])

        @pl.loop(0, NVEC_)
        def _comb(j):
            slot = lane + j * LANES_
            vidx = plsc.load_gather(considx, [slot + s * P_])
            dvec = vidx >> 16
            rv = plsc.load_gather(resp16, [dvec * P_ + slot])
            acc = jnp.where(dvec < NS_, rv, jnp.float32(0.0))
            plsc.store_scatter(outbuf, [slot], acc)
        pltpu.sync_copy(outbuf, out_hbm.at[c, pl.ds(base, P_)])


def kernel(positions, importances, grid_values):
    n = positions.shape[0]
    pos_flat = positions.reshape(-1)
    imp_flat = importances.reshape(-1)
    gv = grid_values.reshape(-1)
    mesh = plsc.VectorSubcoreMesh(core_axis_name="c", subcore_axis_name="s",
                                  num_cores=NC_, num_subcores=NS_)
    out, _ = pl.kernel(
        functools.partial(_sc_body, n),
        out_type=(jax.ShapeDtypeStruct((NC_, n), jnp.float32),
                  jax.ShapeDtypeStruct((NC_, n), jnp.int32)),
        mesh=mesh,
        compiler_params=pltpu.CompilerParams(needs_layout_passes=False),
        scratch_types=[
            pltpu.VMEM((P_ * 3,), jnp.float32),
            pltpu.VMEM((P_,), jnp.int32),
            pltpu.VMEM((NS_ * P_,), jnp.int32),
            pltpu.VMEM((NS_ * P_,), jnp.float32),
            pltpu.VMEM(((NS_ + 1) * P_,), jnp.float32),
            pltpu.VMEM((P_,), jnp.float32),
            pltpu.VMEM((TILEG_ + LANES_,), jnp.float32),
            pltpu.VMEM_SHARED((2 * RP_,), jnp.int32),
            pltpu.VMEM_SHARED((2 * RP_,), jnp.float32),
            pltpu.VMEM_SHARED((2 * NS_ * RP_,), jnp.float32),
            pltpu.SemaphoreType.DMA,
        ],
    )(pos_flat, imp_flat, gv)
    return (out[0] + out[1]).reshape(n, 1)


# speedup vs baseline: 24.5437x; 1.0087x over previous
import functools

import jax
import jax.numpy as jnp
from jax import lax
from jax.experimental import pallas as pl
from jax.experimental.pallas import tpu as pltpu
from jax.experimental.pallas import tpu_sc as plsc

RES_ = 128
GRID_ = RES_ * RES_ * RES_
NC_ = 2
NS_ = 16
LANES_ = 16
HALF_ = GRID_ // NC_
TILEG_ = HALF_ // NS_
P_ = 512
NVEC_ = P_ // LANES_
RP_ = NS_ * P_


def _voxelize(p):
    u = (p + 1.0) * jnp.float32(0.5) * jnp.float32(RES_)
    i = u.astype(jnp.int32)
    return jnp.clip(i, 0, RES_ - 1)


def _sc_body(n_total, pos_hbm, imp_hbm, gv_hbm, out_hbm, idxscr_hbm,
             posbuf, prodidx, considx, consval, resp16, outbuf, subgrid,
             sp_idx, sp_val, sp_resp, sem):
    c = lax.axis_index("c")
    s = lax.axis_index("s")
    chunk = n_total // NS_
    rounds = chunk // P_
    half_lo = c * HALF_
    lane = jnp.arange(LANES_, dtype=jnp.int32)
    lane3 = lane * 3

    pltpu.sync_copy(gv_hbm.at[pl.ds(half_lo + s * TILEG_, TILEG_)],
                    subgrid.at[pl.ds(0, TILEG_)])
    subgrid[pl.ds(TILEG_, LANES_)] = jnp.zeros((LANES_,), jnp.float32)
    plsc.subcore_barrier()

    chunkbase = s * chunk

    @pl.loop(0, rounds)
    def _p1(r):
        par = (r % 2) * RP_
        base = chunkbase + r * P_
        cp_imp = pltpu.async_copy(imp_hbm.at[pl.ds(base, P_)],
                                  sp_val.at[pl.ds(par + s * P_, P_)], sem)
        cp_pos = pltpu.async_copy(pos_hbm.at[pl.ds(base * 3, P_ * 3)],
                                  posbuf, sem)
        cp_pos.wait()

        @pl.loop(0, NVEC_, unroll=8)
        def _vec(j):
            off = j * (3 * LANES_)
            x = plsc.load_gather(posbuf, [lane3 + off])
            y = plsc.load_gather(posbuf, [lane3 + (off + 1)])
            z = plsc.load_gather(posbuf, [lane3 + (off + 2)])
            lin = (_voxelize(x) * RES_ + _voxelize(y)) * RES_ + _voxelize(z)
            keep = (lin >= half_lo) & (lin < half_lo + HALF_)
            plsc.store_scatter(prodidx, [lane + j * LANES_],
                               jnp.where(keep, lin - half_lo, HALF_))

        cp_scr = pltpu.async_copy(
            prodidx, idxscr_hbm.at[c, pl.ds(r * RP_ + s * P_, P_)], sem)
        pltpu.sync_copy(prodidx, sp_idx.at[pl.ds(par + s * P_, P_)])
        cp_scr.wait()
        cp_imp.wait()
        plsc.subcore_barrier()

        pltpu.sync_copy(sp_idx.at[pl.ds(par, RP_)], considx)
        pltpu.sync_copy(sp_val.at[pl.ds(par, RP_)], consval)

        @pl.loop(0, NS_ * NVEC_, unroll=8)
        def _add(j):
            vidx = plsc.load_gather(considx, [lane + j * LANES_])
            vval = plsc.load_gather(consval, [lane + j * LANES_])
            own = (vidx >> 16) == s
            plsc.addupdate_scatter(subgrid, [vidx & (TILEG_ - 1)], vval,
                                   mask=own)

    plsc.subcore_barrier()

    @pl.loop(0, rounds)
    def _p2(r):
        par = (r % 2) * (NS_ * RP_)
        base = chunkbase + r * P_
        pltpu.sync_copy(idxscr_hbm.at[c, pl.ds(r * RP_, RP_)], considx)

        @pl.loop(0, NS_ * NVEC_, unroll=8)
        def _ans(j):
            vidx = plsc.load_gather(considx, [lane + j * LANES_])
            own = (vidx >> 16) == s
            lidx = jnp.where(own, vidx & (TILEG_ - 1), TILEG_)
            vals = plsc.load_gather(subgrid, [lidx])
            plsc.store_scatter(consval, [lane + j * LANES_], vals)

        copies = []
        for p in range(NS_):
            copies.append(pltpu.async_copy(
                consval.at[pl.ds(p * P_, P_)],
                sp_resp.at[pl.ds(par + (p * NS_ + s) * P_, P_)], sem))
        for cp in copies:
            cp.wait()
        plsc.subcore_barrier()

        pltpu.sync_copy(sp_resp.at[pl.ds(par + s * NS_ * P_, NS_ * P_)],
                        resp16.at[pl.ds(0, NS_ * P_)])

        @pl.loop(0, NVEC_, unroll=8)
        def _comb(j):
            slot = lane + j * LANES_
            vidx = plsc.load_gather(considx, [slot + s * P_])
            dvec = vidx >> 16
            rv = plsc.load_gather(resp16, [dvec * P_ + slot])
            acc = jnp.where(dvec < NS_, rv, jnp.float32(0.0))
            plsc.store_scatter(outbuf, [slot], acc)
        pltpu.sync_copy(outbuf, out_hbm.at[c, pl.ds(base, P_)])


def kernel(positions, importances, grid_values):
    n = positions.shape[0]
    pos_flat = positions.reshape(-1)
    imp_flat = importances.reshape(-1)
    gv = grid_values.reshape(-1)
    mesh = plsc.VectorSubcoreMesh(core_axis_name="c", subcore_axis_name="s",
                                  num_cores=NC_, num_subcores=NS_)
    out, _ = pl.kernel(
        functools.partial(_sc_body, n),
        out_type=(jax.ShapeDtypeStruct((NC_, n), jnp.float32),
                  jax.ShapeDtypeStruct((NC_, n), jnp.int32)),
        mesh=mesh,
        compiler_params=pltpu.CompilerParams(needs_layout_passes=False),
        scratch_types=[
            pltpu.VMEM((P_ * 3,), jnp.float32),
            pltpu.VMEM((P_,), jnp.int32),
            pltpu.VMEM((NS_ * P_,), jnp.int32),
            pltpu.VMEM((NS_ * P_,), jnp.float32),
            pltpu.VMEM(((NS_ + 1) * P_,), jnp.float32),
            pltpu.VMEM((P_,), jnp.float32),
            pltpu.VMEM((TILEG_ + LANES_,), jnp.float32),
            pltpu.VMEM_SHARED((2 * RP_,), jnp.int32),
            pltpu.VMEM_SHARED((2 * RP_,), jnp.float32),
            pltpu.VMEM_SHARED((2 * NS_ * RP_,), jnp.float32),
            pltpu.SemaphoreType.DMA,
        ],
    )(pos_flat, imp_flat, gv)
    return (out[0] + out[1]).reshape(n, 1)


# speedup vs baseline: 25.5923x; 1.0427x over previous
import functools

import jax
import jax.numpy as jnp
from jax import lax
from jax.experimental import pallas as pl
from jax.experimental.pallas import tpu as pltpu
from jax.experimental.pallas import tpu_sc as plsc

RES_ = 128
GRID_ = RES_ * RES_ * RES_
NC_ = 2
NS_ = 16
LANES_ = 16
HALF_ = GRID_ // NC_
TILEG_ = HALF_ // NS_
P_ = 1024
NVEC_ = P_ // LANES_
RP_ = NS_ * P_
HB_ = RP_ // 2
HV_ = HB_ // LANES_


def _voxelize(p):
    u = (p + 1.0) * jnp.float32(0.5) * jnp.float32(RES_)
    i = u.astype(jnp.int32)
    return jnp.clip(i, 0, RES_ - 1)


def _sc_body(n_total, pos_hbm, imp_hbm, gv_hbm, out_hbm, idxscr_hbm,
             posbuf, prodidx, considx, consval, resp8, outbuf, subgrid,
             sp_idx, sp_val, sp_resp,
             sem_pos, sem_imp, sem_scr, sem_ci, sem_cv, sem_h0, sem_h1,
             sem_resp):
    c = lax.axis_index("c")
    s = lax.axis_index("s")
    chunk = n_total // NS_
    rounds = chunk // P_
    half_lo = c * HALF_
    lane = jnp.arange(LANES_, dtype=jnp.int32)
    lane3 = lane * 3

    pltpu.sync_copy(gv_hbm.at[pl.ds(half_lo + s * TILEG_, TILEG_)],
                    subgrid.at[pl.ds(0, TILEG_)])
    subgrid[pl.ds(TILEG_, LANES_)] = jnp.zeros((LANES_,), jnp.float32)
    plsc.subcore_barrier()

    chunkbase = s * chunk

    def pos_copy(r, pb):
        return pltpu.make_async_copy(
            pos_hbm.at[pl.ds((chunkbase + r * P_) * 3, P_ * 3)],
            posbuf.at[pl.ds(pb * (P_ * 3), P_ * 3)], sem_pos)

    def imp_copy(r):
        return pltpu.make_async_copy(
            imp_hbm.at[pl.ds(chunkbase + r * P_, P_)],
            sp_val.at[pl.ds((r % 3) * RP_ + s * P_, P_)], sem_imp)

    pos_copy(0, 0).start()
    imp_copy(0).start()

    @pl.loop(0, rounds)
    def _p1(r):
        pb = r % 2
        par = (r % 2) * RP_
        pos_copy(r, pb).wait()

        @pl.loop(0, NVEC_, unroll=8)
        def _vec(j):
            off = pb * (P_ * 3) + j * (3 * LANES_)
            x = plsc.load_gather(posbuf, [lane3 + off])
            y = plsc.load_gather(posbuf, [lane3 + (off + 1)])
            z = plsc.load_gather(posbuf, [lane3 + (off + 2)])
            lin = (_voxelize(x) * RES_ + _voxelize(y)) * RES_ + _voxelize(z)
            keep = (lin >= half_lo) & (lin < half_lo + HALF_)
            plsc.store_scatter(prodidx, [lane + j * LANES_],
                               jnp.where(keep, lin - half_lo, HALF_))

        cp_scr = pltpu.async_copy(
            prodidx, idxscr_hbm.at[c, pl.ds(r * RP_ + s * P_, P_)], sem_scr)

        @pl.when(r + 1 < rounds)
        def _prefetch():
            pos_copy(r + 1, 1 - pb).start()
            imp_copy(r + 1).start()

        pltpu.sync_copy(prodidx, sp_idx.at[pl.ds(par + s * P_, P_)])
        cp_scr.wait()
        imp_copy(r).wait()
        plsc.subcore_barrier()

        cp_ci = pltpu.async_copy(sp_idx.at[pl.ds(par, RP_)], considx, sem_ci)
        vpar = (r % 3) * RP_
        cp_cv = pltpu.async_copy(sp_val.at[pl.ds(vpar, HB_)],
                                 consval, sem_cv)
        cp_ci.wait()
        cp_cv.wait()

        def add_half(h):
            @pl.loop(0, HV_, unroll=8)
            def _add(j):
                e = h * HB_ + j * LANES_
                vidx = plsc.load_gather(considx, [lane + e])
                vval = plsc.load_gather(consval, [lane + j * LANES_])
                own = (vidx >> 16) == s
                plsc.addupdate_scatter(subgrid, [vidx & (TILEG_ - 1)], vval,
                                       mask=own)

        add_half(0)
        pltpu.sync_copy(sp_val.at[pl.ds(vpar + HB_, HB_)], consval)
        add_half(1)

    plsc.subcore_barrier()

    @pl.loop(0, rounds)
    def _p2(r):
        base = chunkbase + r * P_
        cp_h0 = pltpu.async_copy(
            idxscr_hbm.at[c, pl.ds(r * RP_, HB_)],
            considx.at[pl.ds(0, HB_)], sem_h0)
        cp_h1 = pltpu.async_copy(
            idxscr_hbm.at[c, pl.ds(r * RP_ + HB_, HB_)],
            considx.at[pl.ds(HB_, HB_)], sem_h1)

        def answer_half(h):
            @pl.loop(0, HV_, unroll=8)
            def _ans(j):
                e = h * HB_ + j * LANES_
                vidx = plsc.load_gather(considx, [lane + e])
                own = (vidx >> 16) == s
                lidx = jnp.where(own, vidx & (TILEG_ - 1), TILEG_)
                vals = plsc.load_gather(subgrid, [lidx])
                plsc.store_scatter(consval, [lane + j * LANES_], vals)
            return [pltpu.async_copy(
                consval.at[pl.ds(p * P_, P_)],
                sp_resp.at[pl.ds(((h * 8 + p) * NS_ + s) * P_, P_)],
                sem_resp) for p in range(8)]

        cp_h0.wait()
        resp_cp = answer_half(0)
        cp_h1.wait()
        for cp in resp_cp:
            cp.wait()
        resp_cp = answer_half(1)
        for cp in resp_cp:
            cp.wait()
        plsc.subcore_barrier()

        for h in range(2):
            pltpu.sync_copy(
                sp_resp.at[pl.ds((s * NS_ + h * 8) * P_, 8 * P_)],
                resp8.at[pl.ds(0, 8 * P_)])

            @pl.loop(0, NVEC_, unroll=8)
            def _comb(j):
                slot = lane + j * LANES_
                vidx = plsc.load_gather(considx, [slot + s * P_])
                dloc = (vidx >> 16) - h * 8
                inh = (dloc >= 0) & (dloc < 8)
                rv = plsc.load_gather(
                    resp8, [jnp.where(inh, dloc, 8) * P_ + slot])
                val = jnp.where(inh, rv, jnp.float32(0.0))
                if h == 0:
                    plsc.store_scatter(outbuf, [slot], val)
                else:
                    plsc.addupdate_scatter(outbuf, [slot], val, mask=inh)

        pltpu.sync_copy(outbuf, out_hbm.at[c, pl.ds(base, P_)])
        plsc.subcore_barrier()


def kernel(positions, importances, grid_values):
    n = positions.shape[0]
    pos_flat = positions.reshape(-1)
    imp_flat = importances.reshape(-1)
    gv = grid_values.reshape(-1)
    mesh = plsc.VectorSubcoreMesh(core_axis_name="c", subcore_axis_name="s",
                                  num_cores=NC_, num_subcores=NS_)
    out, _ = pl.kernel(
        functools.partial(_sc_body, n),
        out_type=(jax.ShapeDtypeStruct((NC_, n), jnp.float32),
                  jax.ShapeDtypeStruct((NC_, n), jnp.int32)),
        mesh=mesh,
        compiler_params=pltpu.CompilerParams(needs_layout_passes=False),
        scratch_types=[
            pltpu.VMEM((2 * P_ * 3,), jnp.float32),
            pltpu.VMEM((P_,), jnp.int32),
            pltpu.VMEM((RP_,), jnp.int32),
            pltpu.VMEM((HB_,), jnp.float32),
            pltpu.VMEM((9 * P_,), jnp.float32),
            pltpu.VMEM((P_,), jnp.float32),
            pltpu.VMEM((TILEG_ + LANES_,), jnp.float32),
            pltpu.VMEM_SHARED((2 * RP_,), jnp.int32),
            pltpu.VMEM_SHARED((3 * RP_,), jnp.float32),
            pltpu.VMEM_SHARED((NS_ * RP_,), jnp.float32),
            pltpu.SemaphoreType.DMA,
            pltpu.SemaphoreType.DMA,
            pltpu.SemaphoreType.DMA,
            pltpu.SemaphoreType.DMA,
            pltpu.SemaphoreType.DMA,
            pltpu.SemaphoreType.DMA,
            pltpu.SemaphoreType.DMA,
            pltpu.SemaphoreType.DMA,
        ],
    )(pos_flat, imp_flat, gv)
    return (out[0] + out[1]).reshape(n, 1)


# speedup vs baseline: 26.3852x; 1.0310x over previous
import functools

import jax
import jax.numpy as jnp
from jax import lax
from jax.experimental import pallas as pl
from jax.experimental.pallas import tpu as pltpu
from jax.experimental.pallas import tpu_sc as plsc

RES_ = 128
GRID_ = RES_ * RES_ * RES_
NC_ = 2
NS_ = 16
LANES_ = 16
HALF_ = GRID_ // NC_
TILEG_ = HALF_ // NS_
P_ = 1024
NVEC_ = P_ // LANES_
RP_ = NS_ * P_
HB_ = RP_ // 2
HV_ = HB_ // LANES_


def _voxelize(p):
    u = (p + 1.0) * jnp.float32(0.5) * jnp.float32(RES_)
    i = u.astype(jnp.int32)
    return jnp.clip(i, 0, RES_ - 1)


def _sc_body(n_total, pos_hbm, imp_hbm, gv_hbm, out_hbm, idxscr_hbm,
             posbuf, prodidx, considx, consval, resp8, outbuf, subgrid,
             sp_idx, sp_val, sp_resp,
             sem_pos, sem_imp, sem_scr, sem_ci, sem_cv, sem_h0, sem_h1,
             sem_resp):
    c = lax.axis_index("c")
    s = lax.axis_index("s")
    chunk = n_total // NS_
    rounds = chunk // P_
    half_lo = c * HALF_
    lane = jnp.arange(LANES_, dtype=jnp.int32)
    lane3 = lane * 3

    pltpu.sync_copy(gv_hbm.at[pl.ds(half_lo + s * TILEG_, TILEG_)],
                    subgrid.at[pl.ds(0, TILEG_)])
    subgrid[pl.ds(TILEG_, LANES_)] = jnp.zeros((LANES_,), jnp.float32)
    plsc.subcore_barrier()

    chunkbase = s * chunk

    def pos_copy(r, pb):
        return pltpu.make_async_copy(
            pos_hbm.at[pl.ds((chunkbase + r * P_) * 3, P_ * 3)],
            posbuf.at[pl.ds(pb * (P_ * 3), P_ * 3)], sem_pos)

    def imp_copy(r):
        return pltpu.make_async_copy(
            imp_hbm.at[pl.ds(chunkbase + r * P_, P_)],
            sp_val.at[pl.ds((r % 3) * RP_ + s * P_, P_)], sem_imp)

    pos_copy(0, 0).start()
    imp_copy(0).start()

    @pl.loop(0, rounds)
    def _p1(r):
        pb = r % 2
        par = (r % 2) * RP_
        pos_copy(r, pb).wait()

        @pl.loop(0, NVEC_, unroll=8)
        def _vec(j):
            off = pb * (P_ * 3) + j * (3 * LANES_)
            x = plsc.load_gather(posbuf, [lane3 + off])
            y = plsc.load_gather(posbuf, [lane3 + (off + 1)])
            z = plsc.load_gather(posbuf, [lane3 + (off + 2)])
            lin = (_voxelize(x) * RES_ + _voxelize(y)) * RES_ + _voxelize(z)
            keep = (lin >= half_lo) & (lin < half_lo + HALF_)
            prodidx[pl.ds(j * LANES_, LANES_)] = (
                jnp.where(keep, lin - half_lo, HALF_))

        cp_scr = pltpu.async_copy(
            prodidx, idxscr_hbm.at[c, pl.ds(r * RP_ + s * P_, P_)], sem_scr)

        @pl.when(r + 1 < rounds)
        def _prefetch():
            pos_copy(r + 1, 1 - pb).start()
            imp_copy(r + 1).start()

        pltpu.sync_copy(prodidx, sp_idx.at[pl.ds(par + s * P_, P_)])
        cp_scr.wait()
        imp_copy(r).wait()
        plsc.subcore_barrier()

        cp_ci = pltpu.async_copy(sp_idx.at[pl.ds(par, RP_)], considx, sem_ci)
        vpar = (r % 3) * RP_
        cp_cv = pltpu.async_copy(sp_val.at[pl.ds(vpar, HB_)],
                                 consval, sem_cv)
        cp_ci.wait()
        cp_cv.wait()

        def add_half(h):
            @pl.loop(0, HV_, unroll=8)
            def _add(j):
                vidx = considx[pl.ds(h * HB_ + j * LANES_, LANES_)]
                vval = consval[pl.ds(j * LANES_, LANES_)]
                own = (vidx >> 16) == s
                plsc.addupdate_scatter(subgrid, [vidx & (TILEG_ - 1)], vval,
                                       mask=own)

        add_half(0)
        pltpu.sync_copy(sp_val.at[pl.ds(vpar + HB_, HB_)], consval)
        add_half(1)

    plsc.subcore_barrier()

    @pl.loop(0, rounds)
    def _p2(r):
        base = chunkbase + r * P_
        cp_h0 = pltpu.async_copy(
            idxscr_hbm.at[c, pl.ds(r * RP_, HB_)],
            considx.at[pl.ds(0, HB_)], sem_h0)
        cp_h1 = pltpu.async_copy(
            idxscr_hbm.at[c, pl.ds(r * RP_ + HB_, HB_)],
            considx.at[pl.ds(HB_, HB_)], sem_h1)

        def answer_half(h):
            @pl.loop(0, HV_, unroll=8)
            def _ans(j):
                vidx = considx[pl.ds(h * HB_ + j * LANES_, LANES_)]
                own = (vidx >> 16) == s
                lidx = jnp.where(own, vidx & (TILEG_ - 1), TILEG_)
                consval[pl.ds(j * LANES_, LANES_)] = (
                    plsc.load_gather(subgrid, [lidx]))
            return [pltpu.async_copy(
                consval.at[pl.ds(p * P_, P_)],
                sp_resp.at[pl.ds(((h * 8 + p) * NS_ + s) * P_, P_)],
                sem_resp) for p in range(8)]

        cp_h0.wait()
        resp_cp = answer_half(0)
        cp_h1.wait()
        for cp in resp_cp:
            cp.wait()
        resp_cp = answer_half(1)
        for cp in resp_cp:
            cp.wait()
        plsc.subcore_barrier()

        for h in range(2):
            pltpu.sync_copy(
                sp_resp.at[pl.ds((s * NS_ + h * 8) * P_, 8 * P_)],
                resp8.at[pl.ds(0, 8 * P_)])

            @pl.loop(0, NVEC_, unroll=8)
            def _comb(j):
                slot = lane + j * LANES_
                vidx = considx[pl.ds(s * P_ + j * LANES_, LANES_)]
                dloc = (vidx >> 16) - h * 8
                inh = (dloc >= 0) & (dloc < 8)
                rv = plsc.load_gather(
                    resp8, [jnp.where(inh, dloc, 8) * P_ + slot])
                val = jnp.where(inh, rv, jnp.float32(0.0))
                if h == 0:
                    outbuf[pl.ds(j * LANES_, LANES_)] = val
                else:
                    prev = outbuf[pl.ds(j * LANES_, LANES_)]
                    outbuf[pl.ds(j * LANES_, LANES_)] = prev + val

        pltpu.sync_copy(outbuf, out_hbm.at[c, pl.ds(base, P_)])
        plsc.subcore_barrier()


def kernel(positions, importances, grid_values):
    n = positions.shape[0]
    pos_flat = positions.reshape(-1)
    imp_flat = importances.reshape(-1)
    gv = grid_values.reshape(-1)
    mesh = plsc.VectorSubcoreMesh(core_axis_name="c", subcore_axis_name="s",
                                  num_cores=NC_, num_subcores=NS_)
    out, _ = pl.kernel(
        functools.partial(_sc_body, n),
        out_type=(jax.ShapeDtypeStruct((NC_, n), jnp.float32),
                  jax.ShapeDtypeStruct((NC_, n), jnp.int32)),
        mesh=mesh,
        compiler_params=pltpu.CompilerParams(needs_layout_passes=False),
        scratch_types=[
            pltpu.VMEM((2 * P_ * 3,), jnp.float32),
            pltpu.VMEM((P_,), jnp.int32),
            pltpu.VMEM((RP_,), jnp.int32),
            pltpu.VMEM((HB_,), jnp.float32),
            pltpu.VMEM((9 * P_,), jnp.float32),
            pltpu.VMEM((P_,), jnp.float32),
            pltpu.VMEM((TILEG_ + LANES_,), jnp.float32),
            pltpu.VMEM_SHARED((2 * RP_,), jnp.int32),
            pltpu.VMEM_SHARED((3 * RP_,), jnp.float32),
            pltpu.VMEM_SHARED((NS_ * RP_,), jnp.float32),
            pltpu.SemaphoreType.DMA,
            pltpu.SemaphoreType.DMA,
            pltpu.SemaphoreType.DMA,
            pltpu.SemaphoreType.DMA,
            pltpu.SemaphoreType.DMA,
            pltpu.SemaphoreType.DMA,
            pltpu.SemaphoreType.DMA,
            pltpu.SemaphoreType.DMA,
        ],
    )(pos_flat, imp_flat, gv)
    return (out[0] + out[1]).reshape(n, 1)
